# baseline (speedup 1.0000x reference)
import jax
import jax.numpy as jnp
from jax.experimental import pallas as pl
from jax.experimental.pallas import tpu as pltpu

THR = 0.9
_BLOCK = 2000


def _body(nidx_ref, score_ref, out_ref):
    n = nidx_ref[...]
    s = score_ref[...]
    ones = jnp.ones((n.shape[0], 1), dtype=jnp.float32)
    full = jnp.concatenate([ones, s], axis=1)
    out_ref[...] = jnp.where(full < THR, -1, n)


def kernel(nidx, score, specweights, tidxs):
    V, K = nidx.shape
    score2 = jax.lax.squeeze(score, dimensions=(2,))
    grid = (V // _BLOCK,)
    return pl.pallas_call(
        _body,
        grid=grid,
        in_specs=[
            pl.BlockSpec((_BLOCK, K), lambda i: (i, 0)),
            pl.BlockSpec((_BLOCK, K - 1), lambda i: (i, 0)),
        ],
        out_specs=pl.BlockSpec((_BLOCK, K), lambda i: (i, 0)),
        out_shape=jax.ShapeDtypeStruct((V, K), jnp.int32),
        compiler_params=pltpu.CompilerParams(
            dimension_semantics=("parallel",),
        ),
    )(nidx, score2)


# speedup vs baseline: 3.6854x; 3.6854x over previous
import jax
import jax.numpy as jnp
from jax.experimental import pallas as pl
from jax.experimental.pallas import tpu as pltpu

THR = 0.9
_BL = 2048


def _body(nidx_ref, score_ref, out_ref):
    n = nidx_ref[...]
    s3 = score_ref[...]
    s = s3.reshape(s3.shape[0], s3.shape[2])
    ones = jnp.ones((1, s.shape[1]), dtype=jnp.float32)
    full = jnp.concatenate([ones, s], axis=0)
    out_ref[...] = jnp.where(full < THR, -1, n)


def kernel(nidx, score, specweights, tidxs):
    V, K = nidx.shape
    nidx_t = nidx.T
    score_t = jnp.transpose(score, (1, 2, 0))
    nb = pl.cdiv(V, _BL)
    out_t = pl.pallas_call(
        _body,
        grid=(nb,),
        in_specs=[
            pl.BlockSpec((K, _BL), lambda i: (0, i)),
            pl.BlockSpec((K - 1, 1, _BL), lambda i: (0, 0, i)),
        ],
        out_specs=pl.BlockSpec((K, _BL), lambda i: (0, i)),
        out_shape=jax.ShapeDtypeStruct((K, V), jnp.int32),
        compiler_params=pltpu.CompilerParams(
            dimension_semantics=("parallel",),
        ),
    )(nidx_t, score_t)
    return out_t.T


# speedup vs baseline: 5.1492x; 1.3972x over previous
import jax
import jax.numpy as jnp
from jax.experimental import pallas as pl
from jax.experimental.pallas import tpu as pltpu

THR = 0.9
_BL = 4096


def _body(nidx_ref, score_ref, out_ref):
    n = nidx_ref[...]
    s3 = score_ref[...]
    s = s3.reshape(s3.shape[0], s3.shape[2])
    ones = jnp.ones((1, s.shape[1]), dtype=jnp.float32)
    full = jnp.concatenate([ones, s], axis=0)
    out_ref[...] = jnp.where(full < THR, -1, n)


def kernel(nidx, score, specweights, tidxs):
    V, K = nidx.shape
    nidx_t = nidx.T
    score_t = jnp.transpose(score, (1, 2, 0))
    nb = pl.cdiv(V, _BL)
    out_t = pl.pallas_call(
        _body,
        grid=(nb,),
        in_specs=[
            pl.BlockSpec((K, _BL), lambda i: (0, i)),
            pl.BlockSpec((K - 1, 1, _BL), lambda i: (0, 0, i)),
        ],
        out_specs=pl.BlockSpec((K, _BL), lambda i: (0, i)),
        out_shape=jax.ShapeDtypeStruct((K, V), jnp.int32),
        compiler_params=pltpu.CompilerParams(
            dimension_semantics=("parallel",),
        ),
    )(nidx_t, score_t)
    return out_t.T


# speedup vs baseline: 6.1369x; 1.1918x over previous
import jax
import jax.numpy as jnp
from jax.experimental import pallas as pl
from jax.experimental.pallas import tpu as pltpu

THR = 0.9
_BL = 8192


def _body(nidx_ref, score_ref, out_ref):
    n = nidx_ref[...]
    s3 = score_ref[...]
    s = s3.reshape(s3.shape[0], s3.shape[2])
    ones = jnp.ones((1, s.shape[1]), dtype=jnp.float32)
    full = jnp.concatenate([ones, s], axis=0)
    out_ref[...] = jnp.where(full < THR, -1, n)


def kernel(nidx, score, specweights, tidxs):
    V, K = nidx.shape
    nidx_t = nidx.T
    score_t = jnp.transpose(score, (1, 2, 0))
    nb = pl.cdiv(V, _BL)
    out_t = pl.pallas_call(
        _body,
        grid=(nb,),
        in_specs=[
            pl.BlockSpec((K, _BL), lambda i: (0, i)),
            pl.BlockSpec((K - 1, 1, _BL), lambda i: (0, 0, i)),
        ],
        out_specs=pl.BlockSpec((K, _BL), lambda i: (0, i)),
        out_shape=jax.ShapeDtypeStruct((K, V), jnp.int32),
        compiler_params=pltpu.CompilerParams(
            dimension_semantics=("parallel",),
        ),
    )(nidx_t, score_t)
    return out_t.T


# speedup vs baseline: 6.5717x; 1.0709x over previous
import jax
import jax.numpy as jnp
from jax.experimental import pallas as pl
from jax.experimental.pallas import tpu as pltpu

THR = 0.9
_BL = 16384


def _body(nidx_ref, score_ref, out_ref):
    n = nidx_ref[...]
    s3 = score_ref[...]
    s = s3.reshape(s3.shape[0], s3.shape[2])
    ones = jnp.ones((1, s.shape[1]), dtype=jnp.float32)
    full = jnp.concatenate([ones, s], axis=0)
    out_ref[...] = jnp.where(full < THR, -1, n)


def kernel(nidx, score, specweights, tidxs):
    V, K = nidx.shape
    nidx_t = nidx.T
    score_t = jnp.transpose(score, (1, 2, 0))
    nb = pl.cdiv(V, _BL)
    out_t = pl.pallas_call(
        _body,
        grid=(nb,),
        in_specs=[
            pl.BlockSpec((K, _BL), lambda i: (0, i)),
            pl.BlockSpec((K - 1, 1, _BL), lambda i: (0, 0, i)),
        ],
        out_specs=pl.BlockSpec((K, _BL), lambda i: (0, i)),
        out_shape=jax.ShapeDtypeStruct((K, V), jnp.int32),
        compiler_params=pltpu.CompilerParams(
            dimension_semantics=("parallel",),
        ),
    )(nidx_t, score_t)
    return out_t.T
